# baseline (speedup 1.0000x reference)
import functools

import jax
import jax.numpy as jnp
from jax import lax
from jax.experimental import pallas as pl
from jax.experimental.pallas import tpu as pltpu
from jax.experimental.pallas import tpu_sc as plsc

N = 10000
G = 16
NSUB = 16
ROWS = 632
NPAD = NSUB * ROWS
DUMMY = NPAD - 1
ECH = 160
EPC = ECH * 128
SCW = 1
SCH = ECH // SCW
SROWS = SCW * 128
EPAD = NSUB * EPC
F32 = jnp.float32

_mesh = plsc.VectorSubcoreMesh(
    core_axis_name="c", subcore_axis_name="s", num_cores=1)
_sc_params = pltpu.CompilerParams(use_tc_tiling_on_sc=False)


@functools.partial(
    pl.kernel,
    out_type=jax.ShapeDtypeStruct((NPAD, G), F32),
    mesh=_mesh,
    compiler_params=_sc_params,
    scratch_types=[
        pltpu.VMEM_SHARED((NPAD, G), F32),
        pltpu.VMEM((SCH, SROWS), jnp.int32),
        pltpu.VMEM((SROWS, G), F32),
        pltpu.VMEM((ROWS, G), F32),
        pltpu.SemaphoreType.DMA,
    ],
)
def _deg_kernel(dst_hbm, deg_hbm, acc_s, dst_v, ones_v, zz_v, ssem):
    w = lax.axis_index("s")
    sl = pl.ds(w * ROWS, ROWS)
    pltpu.sync_copy(dst_hbm.at[w], dst_v)
    one = jnp.ones((G,), F32)
    zero = jnp.zeros((G,), F32)

    @pl.loop(0, SROWS)
    def _(i):
        ones_v[i] = one

    @pl.loop(0, ROWS)
    def _(i):
        zz_v[i] = zero

    pltpu.sync_copy(zz_v, acc_s.at[sl])
    plsc.subcore_barrier()

    @pl.loop(0, SCH)
    def _(j):
        pltpu.async_copy(ones_v, acc_s.at[dst_v.at[j]], ssem, add=True)

    @pl.loop(0, SCH)
    def _(j):
        pltpu.make_async_copy(ones_v, acc_s.at[dst_v.at[j]], ssem).wait()

    plsc.subcore_barrier()
    pltpu.sync_copy(acc_s.at[sl], deg_hbm.at[sl])


@functools.partial(
    pl.kernel,
    out_type=[jax.ShapeDtypeStruct((NPAD, G), F32)] * 3,
    mesh=_mesh,
    compiler_params=_sc_params,
    scratch_types=[
        pltpu.VMEM_SHARED((NPAD, G), F32),
        pltpu.VMEM_SHARED((NPAD, G), F32),
        pltpu.VMEM((SCH, SROWS), jnp.int32),
        pltpu.VMEM((SCH, SROWS), jnp.int32),
        pltpu.VMEM((SROWS, G), F32),
        pltpu.VMEM((SROWS, G), F32),
        pltpu.VMEM((ROWS, G), F32),
        pltpu.VMEM((ROWS, G), F32),
        pltpu.VMEM((ROWS, G), F32),
        pltpu.SemaphoreType.DMA,
        pltpu.SemaphoreType.DMA,
    ],
)
def _prop_kernel(u0_hbm, d2_hbm, dst_hbm, src_hbm, u2_hbm, u4_hbm, u6_hbm,
                 u_s, acc_s, dst_v, src_v, rows_a, rows_b, a_v, u_v, d_v,
                 ga, gb):
    def dsl(j):
        return dst_v.at[j]

    def ssl(j):
        return src_v.at[j]

    w = lax.axis_index("s")
    sl = pl.ds(w * ROWS, ROWS)
    pltpu.sync_copy(dst_hbm.at[w], dst_v)
    pltpu.sync_copy(src_hbm.at[w], src_v)
    pltpu.sync_copy(d2_hbm.at[sl], d_v)
    pltpu.sync_copy(u0_hbm.at[sl], u_s.at[sl])
    zero = jnp.zeros((G,), F32)

    @pl.loop(0, ROWS)
    def _(i):
        u_v[i] = zero

    pltpu.sync_copy(u_v, acc_s.at[sl])
    plsc.subcore_barrier()

    for r in range(6):
        pltpu.async_copy(u_s.at[dsl(0)], rows_a, ga)

        @pl.loop(0, SCH - 3, step=2)
        def _(j):
            pltpu.async_copy(u_s.at[dsl(j + 1)], rows_b, gb)
            pltpu.make_async_copy(u_s.at[dsl(j)], rows_a, ga).wait()
            pltpu.sync_copy(rows_a, acc_s.at[ssl(j)], add=True)
            pltpu.async_copy(u_s.at[dsl(j + 2)], rows_a, ga)
            pltpu.make_async_copy(u_s.at[dsl(j + 1)], rows_b, gb).wait()
            pltpu.sync_copy(rows_b, acc_s.at[ssl(j + 1)], add=True)

        pltpu.async_copy(u_s.at[dsl(SCH - 1)], rows_b, gb)
        pltpu.make_async_copy(u_s.at[dsl(SCH - 2)], rows_a, ga).wait()
        pltpu.sync_copy(rows_a, acc_s.at[ssl(SCH - 2)], add=True)
        pltpu.make_async_copy(u_s.at[dsl(SCH - 1)], rows_b, gb).wait()
        pltpu.sync_copy(rows_b, acc_s.at[ssl(SCH - 1)], add=True)

        plsc.subcore_barrier()
        pltpu.sync_copy(acc_s.at[sl], a_v)
        pltpu.sync_copy(u_s.at[sl], u_v)

        @pl.loop(0, ROWS, step=4)
        def _(i):
            for t in range(4):
                u_v[i + t] = d_v[i + t] * (a_v[i + t] + u_v[i + t])

        pltpu.sync_copy(u_v, u_s.at[sl])
        if r == 1:
            pltpu.sync_copy(u_v, u2_hbm.at[sl])
        elif r == 3:
            pltpu.sync_copy(u_v, u4_hbm.at[sl])
        elif r == 5:
            pltpu.sync_copy(u_v, u6_hbm.at[sl])
        if r < 5:
            @pl.loop(0, ROWS)
            def _(i):
                u_v[i] = zero

            pltpu.sync_copy(u_v, acc_s.at[sl])
        plsc.subcore_barrier()


def _prep_body(degm1_ref, batch_ref, u0_ref, d2_ref, sqd_ref, cind_ref):
    deg = degm1_ref[...] + 1.0
    gid = lax.broadcasted_iota(jnp.int32, (1, G), 1)
    onehot = (batch_ref[...] == gid).astype(F32)
    counts = jnp.sum(onehot, axis=0, keepdims=True)
    cmax = jnp.maximum(counts, 1.0)
    u0_ref[...] = onehot * lax.rsqrt(deg) / cmax
    d2_ref[...] = 1.0 / deg
    sqd_ref[...] = jnp.sqrt(deg)
    cind_ref[...] = counts / cmax


_prep_call = pl.pallas_call(
    _prep_body,
    out_shape=(
        jax.ShapeDtypeStruct((NPAD, G), F32),
        jax.ShapeDtypeStruct((NPAD, G), F32),
        jax.ShapeDtypeStruct((NPAD, G), F32),
        jax.ShapeDtypeStruct((1, G), F32),
    ),
)


def _epi_body(u2_ref, u4_ref, u6_ref, sqd_ref, x_ref, cind_ref,
              W1_ref, b1_ref, W2_ref, b2_ref, W3_ref, b3_ref,
              fcW_ref, fcb_ref, out_ref):
    f32 = dict(preferred_element_type=F32)
    sqd = sqd_ref[...]
    s2 = jnp.sum(sqd * u2_ref[...], axis=0, keepdims=True)
    s4 = jnp.sum(sqd * u4_ref[...], axis=0, keepdims=True)
    q6 = (sqd * u6_ref[...])[:N, :]
    q6x = lax.dot_general(q6, x_ref[...], (((0,), (0,)), ((), ())), **f32)
    W3 = W3_ref[...]
    W32 = jnp.dot(W3, W2_ref[...], **f32)
    Wc = jnp.dot(W32, W1_ref[...], **f32)
    w21 = lax.dot_general(b1_ref[...], W32, (((1,), (1,)), ((), ())), **f32)
    wb2 = lax.dot_general(b2_ref[...], W3, (((1,), (1,)), ((), ())), **f32)
    outer = (((0,), (0,)), ((), ()))
    pooled = (lax.dot_general(q6x, Wc, (((1,), (1,)), ((), ())), **f32)
              + lax.dot_general(s4, w21, outer, **f32)
              + lax.dot_general(s2, wb2, outer, **f32)
              + lax.dot_general(cind_ref[...], b3_ref[...], outer, **f32))
    logits = lax.dot_general(pooled, fcW_ref[...],
                             (((1,), (1,)), ((), ())), **f32) + fcb_ref[...]
    m = jnp.max(logits, axis=1, keepdims=True)
    lse = jnp.log(jnp.sum(jnp.exp(logits - m), axis=1, keepdims=True)) + m
    out_ref[...] = logits - lse


_epi_call = pl.pallas_call(
    _epi_body,
    out_shape=jax.ShapeDtypeStruct((G, 10), F32),
)


def kernel(x, edge_index, batch, W1, b1, W2, b2, W3, b3, fcW, fcb):
    src = edge_index[0]
    dst = edge_index[1]
    pad = jnp.full((EPAD - src.shape[0],), DUMMY, jnp.int32)
    srcg = jnp.concatenate([src, pad]).reshape(NSUB, SCH, SROWS)
    dstg = jnp.concatenate([dst, pad]).reshape(NSUB, SCH, SROWS)
    batchp = jnp.concatenate(
        [batch, jnp.full((NPAD - N,), G, jnp.int32)]).reshape(NPAD, 1)

    degm1 = _deg_kernel(dstg)
    u0, d2, sqd, cind = _prep_call(degm1, batchp)
    u2, u4, u6 = _prop_kernel(u0, d2, dstg, srcg)
    return _epi_call(u2, u4, u6, sqd, x, cind,
                     W1, b1.reshape(1, -1), W2, b2.reshape(1, -1),
                     W3, b3.reshape(1, -1), fcW, fcb.reshape(1, -1))


# speedup vs baseline: 1.0399x; 1.0399x over previous
import functools

import jax
import jax.numpy as jnp
from jax import lax
from jax.experimental import pallas as pl
from jax.experimental.pallas import tpu as pltpu
from jax.experimental.pallas import tpu_sc as plsc

N = 10000
G = 16
NSUB = 16
ROWS = 632
NPAD = NSUB * ROWS
DUMMY = NPAD - 1
ECH = 160
EPC = ECH * 128
SCW = 1
SCH = ECH // SCW
SROWS = SCW * 128
EPAD = NSUB * EPC
F32 = jnp.float32

_mesh = plsc.VectorSubcoreMesh(
    core_axis_name="c", subcore_axis_name="s", num_cores=1)
_sc_params = pltpu.CompilerParams(use_tc_tiling_on_sc=False)


@functools.partial(
    pl.kernel,
    out_type=jax.ShapeDtypeStruct((NPAD, G), F32),
    mesh=_mesh,
    compiler_params=_sc_params,
    scratch_types=[
        pltpu.VMEM_SHARED((NPAD, G), F32),
        pltpu.VMEM((SCH, SROWS), jnp.int32),
        pltpu.VMEM((SROWS, G), F32),
        pltpu.VMEM((ROWS, G), F32),
        pltpu.SemaphoreType.DMA,
    ],
)
def _deg_kernel(dst_hbm, deg_hbm, acc_s, dst_v, ones_v, zz_v, ssem):
    w = lax.axis_index("s")
    sl = pl.ds(w * ROWS, ROWS)
    pltpu.sync_copy(dst_hbm.at[w], dst_v)
    one = jnp.ones((G,), F32)
    zero = jnp.zeros((G,), F32)

    @pl.loop(0, SROWS)
    def _(i):
        ones_v[i] = one

    @pl.loop(0, ROWS)
    def _(i):
        zz_v[i] = zero

    pltpu.sync_copy(zz_v, acc_s.at[sl])
    plsc.subcore_barrier()

    @pl.loop(0, SCH)
    def _(j):
        pltpu.async_copy(ones_v, acc_s.at[dst_v.at[j]], ssem, add=True)

    @pl.loop(0, SCH)
    def _(j):
        pltpu.make_async_copy(ones_v, acc_s.at[dst_v.at[j]], ssem).wait()

    plsc.subcore_barrier()
    pltpu.sync_copy(acc_s.at[sl], deg_hbm.at[sl])


@functools.partial(
    pl.kernel,
    out_type=[jax.ShapeDtypeStruct((NPAD, G), F32)] * 3,
    mesh=_mesh,
    compiler_params=_sc_params,
    scratch_types=[
        pltpu.VMEM_SHARED((NPAD, G), F32),
        pltpu.VMEM_SHARED((NPAD, G), F32),
        pltpu.VMEM((SCH, SROWS), jnp.int32),
        pltpu.VMEM((SCH, SROWS), jnp.int32),
        pltpu.VMEM((SROWS, G), F32),
        pltpu.VMEM((SROWS, G), F32),
        pltpu.VMEM((ROWS, G), F32),
        pltpu.VMEM((ROWS, G), F32),
        pltpu.VMEM((ROWS, G), F32),
        pltpu.VMEM((ROWS, G), F32),
        pltpu.SemaphoreType.DMA,
        pltpu.SemaphoreType.DMA,
    ],
)
def _prop_kernel(u0_hbm, d2_hbm, dst_hbm, src_hbm, u2_hbm, u4_hbm, u6_hbm,
                 u_s, acc_s, dst_v, src_v, rows_a, rows_b, a_v, u_v, d_v,
                 zz_v, ga, gb):
    def dsl(j):
        return dst_v.at[j]

    def ssl(j):
        return src_v.at[j]

    w = lax.axis_index("s")
    sl = pl.ds(w * ROWS, ROWS)
    pltpu.sync_copy(dst_hbm.at[w], dst_v)
    pltpu.sync_copy(src_hbm.at[w], src_v)
    pltpu.sync_copy(d2_hbm.at[sl], d_v)
    pltpu.sync_copy(u0_hbm.at[sl], u_s.at[sl])
    zero = jnp.zeros((G,), F32)

    @pl.loop(0, ROWS)
    def _(i):
        zz_v[i] = zero

    pltpu.sync_copy(zz_v, acc_s.at[sl])
    plsc.subcore_barrier()

    for r in range(6):
        pltpu.async_copy(u_s.at[dsl(0)], rows_a, ga)

        @pl.loop(0, SCH - 3, step=2)
        def _(j):
            pltpu.async_copy(u_s.at[dsl(j + 1)], rows_b, gb)
            pltpu.make_async_copy(u_s.at[dsl(j)], rows_a, ga).wait()
            pltpu.sync_copy(rows_a, acc_s.at[ssl(j)], add=True)
            pltpu.async_copy(u_s.at[dsl(j + 2)], rows_a, ga)
            pltpu.make_async_copy(u_s.at[dsl(j + 1)], rows_b, gb).wait()
            pltpu.sync_copy(rows_b, acc_s.at[ssl(j + 1)], add=True)

        pltpu.async_copy(u_s.at[dsl(SCH - 1)], rows_b, gb)
        pltpu.make_async_copy(u_s.at[dsl(SCH - 2)], rows_a, ga).wait()
        pltpu.sync_copy(rows_a, acc_s.at[ssl(SCH - 2)], add=True)
        pltpu.make_async_copy(u_s.at[dsl(SCH - 1)], rows_b, gb).wait()
        pltpu.sync_copy(rows_b, acc_s.at[ssl(SCH - 1)], add=True)

        plsc.subcore_barrier()
        pltpu.sync_copy(acc_s.at[sl], a_v)
        pltpu.sync_copy(u_s.at[sl], u_v)

        @pl.loop(0, ROWS, step=4)
        def _(i):
            for t in range(4):
                u_v[i + t] = d_v[i + t] * (a_v[i + t] + u_v[i + t])

        pltpu.sync_copy(u_v, u_s.at[sl])
        if r == 1:
            pltpu.sync_copy(u_v, u2_hbm.at[sl])
        elif r == 3:
            pltpu.sync_copy(u_v, u4_hbm.at[sl])
        elif r == 5:
            pltpu.sync_copy(u_v, u6_hbm.at[sl])
        if r < 5:
            pltpu.sync_copy(zz_v, acc_s.at[sl])
        plsc.subcore_barrier()


def _prep_body(degm1_ref, batch_ref, u0_ref, d2_ref, sqd_ref, cind_ref):
    deg = degm1_ref[...] + 1.0
    gid = lax.broadcasted_iota(jnp.int32, (1, G), 1)
    onehot = (batch_ref[...] == gid).astype(F32)
    counts = jnp.sum(onehot, axis=0, keepdims=True)
    cmax = jnp.maximum(counts, 1.0)
    u0_ref[...] = onehot * lax.rsqrt(deg) / cmax
    d2_ref[...] = 1.0 / deg
    sqd_ref[...] = jnp.sqrt(deg)
    cind_ref[...] = counts / cmax


_prep_call = pl.pallas_call(
    _prep_body,
    out_shape=(
        jax.ShapeDtypeStruct((NPAD, G), F32),
        jax.ShapeDtypeStruct((NPAD, G), F32),
        jax.ShapeDtypeStruct((NPAD, G), F32),
        jax.ShapeDtypeStruct((1, G), F32),
    ),
)


def _epi_body(u2_ref, u4_ref, u6_ref, sqd_ref, x_ref, cind_ref,
              W1_ref, b1_ref, W2_ref, b2_ref, W3_ref, b3_ref,
              fcW_ref, fcb_ref, out_ref):
    f32 = dict(preferred_element_type=F32)
    sqd = sqd_ref[...]
    s2 = jnp.sum(sqd * u2_ref[...], axis=0, keepdims=True)
    s4 = jnp.sum(sqd * u4_ref[...], axis=0, keepdims=True)
    q6 = (sqd * u6_ref[...])[:N, :]
    q6x = lax.dot_general(q6, x_ref[...], (((0,), (0,)), ((), ())), **f32)
    W3 = W3_ref[...]
    W32 = jnp.dot(W3, W2_ref[...], **f32)
    Wc = jnp.dot(W32, W1_ref[...], **f32)
    w21 = lax.dot_general(b1_ref[...], W32, (((1,), (1,)), ((), ())), **f32)
    wb2 = lax.dot_general(b2_ref[...], W3, (((1,), (1,)), ((), ())), **f32)
    outer = (((0,), (0,)), ((), ()))
    pooled = (lax.dot_general(q6x, Wc, (((1,), (1,)), ((), ())), **f32)
              + lax.dot_general(s4, w21, outer, **f32)
              + lax.dot_general(s2, wb2, outer, **f32)
              + lax.dot_general(cind_ref[...], b3_ref[...], outer, **f32))
    logits = lax.dot_general(pooled, fcW_ref[...],
                             (((1,), (1,)), ((), ())), **f32) + fcb_ref[...]
    m = jnp.max(logits, axis=1, keepdims=True)
    lse = jnp.log(jnp.sum(jnp.exp(logits - m), axis=1, keepdims=True)) + m
    out_ref[...] = logits - lse


_epi_call = pl.pallas_call(
    _epi_body,
    out_shape=jax.ShapeDtypeStruct((G, 10), F32),
)


def kernel(x, edge_index, batch, W1, b1, W2, b2, W3, b3, fcW, fcb):
    src = edge_index[0]
    dst = edge_index[1]
    pad = jnp.full((EPAD - src.shape[0],), DUMMY, jnp.int32)
    srcg = jnp.concatenate([src, pad]).reshape(NSUB, SCH, SROWS)
    dstg = jnp.concatenate([dst, pad]).reshape(NSUB, SCH, SROWS)
    batchp = jnp.concatenate(
        [batch, jnp.full((NPAD - N,), G, jnp.int32)]).reshape(NPAD, 1)

    degm1 = _deg_kernel(dstg)
    u0, d2, sqd, cind = _prep_call(degm1, batchp)
    u2, u4, u6 = _prop_kernel(u0, d2, dstg, srcg)
    return _epi_call(u2, u4, u6, sqd, x, cind,
                     W1, b1.reshape(1, -1), W2, b2.reshape(1, -1),
                     W3, b3.reshape(1, -1), fcW, fcb.reshape(1, -1))


# speedup vs baseline: 1.1239x; 1.0807x over previous
import functools

import jax
import jax.numpy as jnp
from jax import lax
from jax.experimental import pallas as pl
from jax.experimental.pallas import tpu as pltpu
from jax.experimental.pallas import tpu_sc as plsc

N = 10000
G = 16
NSUB = 16
ROWS = 632
NPAD = NSUB * ROWS
DUMMY = NPAD - 1
ECH = 160
EPC = ECH * 128
SCW = 1
SCH = ECH // SCW
SROWS = SCW * 128
EPAD = NSUB * EPC
F32 = jnp.float32

_mesh = plsc.VectorSubcoreMesh(
    core_axis_name="c", subcore_axis_name="s", num_cores=1)
_sc_params = pltpu.CompilerParams(use_tc_tiling_on_sc=False)


@functools.partial(
    pl.kernel,
    out_type=jax.ShapeDtypeStruct((NPAD, G), F32),
    mesh=_mesh,
    compiler_params=_sc_params,
    scratch_types=[
        pltpu.VMEM_SHARED((NPAD, G), F32),
        pltpu.VMEM((SCH, SROWS), jnp.int32),
        pltpu.VMEM((SROWS, G), F32),
        pltpu.VMEM((ROWS, G), F32),
        pltpu.SemaphoreType.DMA,
    ],
)
def _deg_kernel(dst_hbm, deg_hbm, acc_s, dst_v, ones_v, zz_v, ssem):
    w = lax.axis_index("s")
    sl = pl.ds(w * ROWS, ROWS)
    pltpu.sync_copy(dst_hbm.at[w], dst_v)
    one = jnp.ones((G,), F32)
    zero = jnp.zeros((G,), F32)

    @pl.loop(0, SROWS)
    def _(i):
        ones_v[i] = one

    @pl.loop(0, ROWS)
    def _(i):
        zz_v[i] = zero

    pltpu.sync_copy(zz_v, acc_s.at[sl])
    plsc.subcore_barrier()

    @pl.loop(0, SCH)
    def _(j):
        pltpu.async_copy(ones_v, acc_s.at[dst_v.at[j]], ssem, add=True)

    @pl.loop(0, SCH)
    def _(j):
        pltpu.make_async_copy(ones_v, acc_s.at[dst_v.at[j]], ssem).wait()

    plsc.subcore_barrier()
    pltpu.sync_copy(acc_s.at[sl], deg_hbm.at[sl])


@functools.partial(
    pl.kernel,
    out_type=[jax.ShapeDtypeStruct((NPAD, G), F32)] * 3,
    mesh=_mesh,
    compiler_params=_sc_params,
    scratch_types=[
        pltpu.VMEM_SHARED((NPAD, G), F32),
        pltpu.VMEM_SHARED((NPAD, G), F32),
        pltpu.VMEM((SCH, SROWS), jnp.int32),
        pltpu.VMEM((SCH, SROWS), jnp.int32),
        pltpu.VMEM((SROWS, G), F32),
        pltpu.VMEM((SROWS, G), F32),
        pltpu.VMEM((ROWS, G), F32),
        pltpu.VMEM((ROWS, G), F32),
        pltpu.VMEM((ROWS, G), F32),
        pltpu.VMEM((ROWS, G), F32),
        pltpu.SemaphoreType.DMA,
        pltpu.SemaphoreType.DMA,
    ],
)
def _prop_kernel(u0_hbm, d2_hbm, dst_hbm, src_hbm, u2_hbm, u4_hbm, u6_hbm,
                 u_s, acc_s, dst_v, src_v, rows_a, rows_b, a_v, u_v, d_v,
                 zz_v, ga, gb):
    def dsl(j):
        return dst_v.at[j]

    def ssl(j):
        return src_v.at[j]

    w = lax.axis_index("s")
    sl = pl.ds(w * ROWS, ROWS)
    pltpu.sync_copy(dst_hbm.at[w], dst_v)
    pltpu.sync_copy(src_hbm.at[w], src_v)
    pltpu.sync_copy(d2_hbm.at[sl], d_v)
    pltpu.sync_copy(u0_hbm.at[sl], u_s.at[sl])
    zero = jnp.zeros((G,), F32)

    @pl.loop(0, ROWS)
    def _(i):
        zz_v[i] = zero

    pltpu.sync_copy(zz_v, acc_s.at[sl])
    plsc.subcore_barrier()

    for r in range(6):
        pltpu.async_copy(u_s.at[dsl(0)], rows_a, ga)

        @pl.loop(0, SCH - 3, step=2)
        def _(j):
            pltpu.async_copy(u_s.at[dsl(j + 1)], rows_b, gb)
            pltpu.make_async_copy(u_s.at[dsl(j)], rows_a, ga).wait()
            pltpu.sync_copy(rows_a, acc_s.at[ssl(j)], add=True)
            pltpu.async_copy(u_s.at[dsl(j + 2)], rows_a, ga)
            pltpu.make_async_copy(u_s.at[dsl(j + 1)], rows_b, gb).wait()
            pltpu.sync_copy(rows_b, acc_s.at[ssl(j + 1)], add=True)

        pltpu.async_copy(u_s.at[dsl(SCH - 1)], rows_b, gb)
        pltpu.make_async_copy(u_s.at[dsl(SCH - 2)], rows_a, ga).wait()
        pltpu.sync_copy(rows_a, acc_s.at[ssl(SCH - 2)], add=True)
        pltpu.make_async_copy(u_s.at[dsl(SCH - 1)], rows_b, gb).wait()
        pltpu.sync_copy(rows_b, acc_s.at[ssl(SCH - 1)], add=True)

        plsc.subcore_barrier()
        pltpu.sync_copy(acc_s.at[sl], a_v)
        pltpu.sync_copy(u_s.at[sl], u_v)

        @pl.loop(0, ROWS, step=4)
        def _(i):
            for t in range(4):
                u_v[i + t] = d_v[i + t] * (a_v[i + t] + u_v[i + t])

        pltpu.sync_copy(u_v, u_s.at[sl])
        if r == 1:
            pltpu.sync_copy(u_v, u2_hbm.at[sl])
        elif r == 3:
            pltpu.sync_copy(u_v, u4_hbm.at[sl])
        elif r == 5:
            pltpu.sync_copy(u_v, u6_hbm.at[sl])
        if r < 5:
            pltpu.sync_copy(zz_v, acc_s.at[sl])
        plsc.subcore_barrier()


def _prep_body(degm1_ref, batch_ref, u0_ref, d2_ref, sqd_ref, cind_ref):
    deg = degm1_ref[...] + 1.0
    gid = lax.broadcasted_iota(jnp.int32, (1, G), 1)
    onehot = (batch_ref[...] == gid).astype(F32)
    counts = jnp.sum(onehot, axis=0, keepdims=True)
    cmax = jnp.maximum(counts, 1.0)
    u0_ref[...] = onehot * lax.rsqrt(deg) / cmax
    d2_ref[...] = 1.0 / deg
    sqd_ref[...] = jnp.sqrt(deg)
    cind_ref[...] = counts / cmax


_prep_call = pl.pallas_call(
    _prep_body,
    out_shape=(
        jax.ShapeDtypeStruct((NPAD, G), F32),
        jax.ShapeDtypeStruct((NPAD, G), F32),
        jax.ShapeDtypeStruct((NPAD, G), F32),
        jax.ShapeDtypeStruct((1, G), F32),
    ),
)


def _epi_body(u2_ref, u4_ref, u6_ref, sqd_ref, x_ref, cind_ref,
              W1_ref, b1_ref, W2_ref, b2_ref, W3_ref, b3_ref,
              fcW_ref, fcb_ref, out_ref):
    f32 = dict(preferred_element_type=F32)
    sqd = sqd_ref[...]
    s2 = jnp.sum(sqd * u2_ref[...], axis=0, keepdims=True)
    s4 = jnp.sum(sqd * u4_ref[...], axis=0, keepdims=True)
    q6 = (sqd * u6_ref[...])[:N, :]
    q6x = lax.dot_general(q6, x_ref[...], (((0,), (0,)), ((), ())), **f32)
    W3 = W3_ref[...]
    W32 = jnp.dot(W3, W2_ref[...], **f32)
    Wc = jnp.dot(W32, W1_ref[...], **f32)
    w21 = lax.dot_general(b1_ref[...], W32, (((1,), (1,)), ((), ())), **f32)
    wb2 = lax.dot_general(b2_ref[...], W3, (((1,), (1,)), ((), ())), **f32)
    outer = (((0,), (0,)), ((), ()))
    pooled = (lax.dot_general(q6x, Wc, (((1,), (1,)), ((), ())), **f32)
              + lax.dot_general(s4, w21, outer, **f32)
              + lax.dot_general(s2, wb2, outer, **f32)
              + lax.dot_general(cind_ref[...], b3_ref[...], outer, **f32))
    logits = lax.dot_general(pooled, fcW_ref[...],
                             (((1,), (1,)), ((), ())), **f32) + fcb_ref[...]
    m = jnp.max(logits, axis=1, keepdims=True)
    lse = jnp.log(jnp.sum(jnp.exp(logits - m), axis=1, keepdims=True)) + m
    out_ref[...] = logits - lse


_epi_call = pl.pallas_call(
    _epi_body,
    out_shape=jax.ShapeDtypeStruct((G, 10), F32),
)


def kernel(x, edge_index, batch, W1, b1, W2, b2, W3, b3, fcW, fcb):
    src = edge_index[0]
    dst = edge_index[1]
    pad = N + jnp.arange(EPAD - src.shape[0], dtype=jnp.int32) % (NPAD - N)
    srcg = jnp.concatenate([src, pad]).reshape(NSUB, SCH, SROWS)
    dstg = jnp.concatenate([dst, pad]).reshape(NSUB, SCH, SROWS)
    batchp = jnp.concatenate(
        [batch, jnp.full((NPAD - N,), G, jnp.int32)]).reshape(NPAD, 1)

    degm1 = _deg_kernel(dstg)
    u0, d2, sqd, cind = _prep_call(degm1, batchp)
    u2, u4, u6 = _prop_kernel(u0, d2, dstg, srcg)
    return _epi_call(u2, u4, u6, sqd, x, cind,
                     W1, b1.reshape(1, -1), W2, b2.reshape(1, -1),
                     W3, b3.reshape(1, -1), fcW, fcb.reshape(1, -1))


# speedup vs baseline: 1.1455x; 1.0192x over previous
import functools

import jax
import jax.numpy as jnp
from jax import lax
from jax.experimental import pallas as pl
from jax.experimental.pallas import tpu as pltpu
from jax.experimental.pallas import tpu_sc as plsc

N = 10000
G = 16
NSUB = 16
ROWS = 632
NPAD = NSUB * ROWS
DUMMY = NPAD - 1
ECH = 160
EPC = ECH * 128
SCW = 1
SCH = ECH // SCW
SROWS = SCW * 128
EPAD = NSUB * EPC
F32 = jnp.float32

_mesh = plsc.VectorSubcoreMesh(
    core_axis_name="c", subcore_axis_name="s", num_cores=1)
_sc_params = pltpu.CompilerParams(
    use_tc_tiling_on_sc=False, needs_layout_passes=False)


@functools.partial(
    pl.kernel,
    out_type=[jax.ShapeDtypeStruct((NPAD, G), F32)] * 4,
    mesh=_mesh,
    compiler_params=_sc_params,
    scratch_types=[
        pltpu.VMEM_SHARED((NPAD, G), F32),
        pltpu.VMEM_SHARED((NPAD, G), F32),
        pltpu.VMEM((SCH, SROWS), jnp.int32),
        pltpu.VMEM((SCH, SROWS), jnp.int32),
        pltpu.VMEM((SROWS, G), F32),
        pltpu.VMEM((SROWS, G), F32),
        pltpu.VMEM((ROWS, G), F32),
        pltpu.VMEM((ROWS, G), F32),
        pltpu.VMEM((ROWS, G), F32),
        pltpu.VMEM((ROWS, G), F32),
        pltpu.SemaphoreType.DMA,
        pltpu.SemaphoreType.DMA,
    ],
)
def _prop_kernel(p0_hbm, dst_hbm, src_hbm, u2_hbm, u4_hbm, u6_hbm, deg_hbm,
                 u_s, acc_s, dst_v, src_v, rows_a, rows_b, a_v, u_v, d_v,
                 zz_v, ga, gb):
    def dsl(j):
        return dst_v.at[j]

    def ssl(j):
        return src_v.at[j]

    w = lax.axis_index("s")
    sl = pl.ds(w * ROWS, ROWS)
    pltpu.sync_copy(dst_hbm.at[w], dst_v)
    pltpu.sync_copy(src_hbm.at[w], src_v)
    zero = jnp.zeros((G,), F32)
    one = jnp.ones((G,), F32)

    @pl.loop(0, ROWS)
    def _(i):
        zz_v[i] = zero

    @pl.loop(0, SROWS)
    def _(i):
        rows_a[i] = one

    pltpu.sync_copy(zz_v, acc_s.at[sl])
    plsc.subcore_barrier()

    @pl.loop(0, SCH)
    def _(j):
        pltpu.async_copy(rows_a, acc_s.at[dsl(j)], ga, add=True)

    @pl.loop(0, SCH)
    def _(j):
        pltpu.make_async_copy(rows_a, acc_s.at[dsl(j)], ga).wait()

    plsc.subcore_barrier()
    pltpu.sync_copy(acc_s.at[sl], a_v)
    pltpu.sync_copy(a_v, deg_hbm.at[sl])
    pltpu.sync_copy(zz_v, acc_s.at[sl])
    pltpu.sync_copy(p0_hbm.at[sl], u_v)
    magic = jnp.full((G,), 0x5F3759DF, jnp.int32)
    shift1 = jnp.full((G,), 1, jnp.int32)
    c15 = jnp.full((G,), 1.5, F32)
    ch = jnp.full((G,), 0.5, F32)

    @pl.loop(0, ROWS)
    def _(i):
        degv = a_v[i] + one
        d_v[i] = one / degv
        y = plsc.bitcast(
            magic - lax.shift_right_logical(plsc.bitcast(degv, jnp.int32),
                                            shift1), F32)
        hx = ch * degv
        y = y * (c15 - hx * y * y)
        y = y * (c15 - hx * y * y)
        y = y * (c15 - hx * y * y)
        u_v[i] = u_v[i] * y

    pltpu.sync_copy(u_v, u_s.at[sl])
    plsc.subcore_barrier()

    for r in range(6):
        pltpu.async_copy(u_s.at[dsl(0)], rows_a, ga)

        @pl.loop(0, SCH - 3, step=2)
        def _(j):
            pltpu.async_copy(u_s.at[dsl(j + 1)], rows_b, gb)
            pltpu.make_async_copy(u_s.at[dsl(j)], rows_a, ga).wait()
            pltpu.sync_copy(rows_a, acc_s.at[ssl(j)], add=True)
            pltpu.async_copy(u_s.at[dsl(j + 2)], rows_a, ga)
            pltpu.make_async_copy(u_s.at[dsl(j + 1)], rows_b, gb).wait()
            pltpu.sync_copy(rows_b, acc_s.at[ssl(j + 1)], add=True)

        pltpu.async_copy(u_s.at[dsl(SCH - 1)], rows_b, gb)
        pltpu.make_async_copy(u_s.at[dsl(SCH - 2)], rows_a, ga).wait()
        pltpu.sync_copy(rows_a, acc_s.at[ssl(SCH - 2)], add=True)
        pltpu.make_async_copy(u_s.at[dsl(SCH - 1)], rows_b, gb).wait()
        pltpu.sync_copy(rows_b, acc_s.at[ssl(SCH - 1)], add=True)

        plsc.subcore_barrier()
        pltpu.sync_copy(acc_s.at[sl], a_v)
        pltpu.sync_copy(u_s.at[sl], u_v)

        @pl.loop(0, ROWS, step=4)
        def _(i):
            for t in range(4):
                u_v[i + t] = d_v[i + t] * (a_v[i + t] + u_v[i + t])

        pltpu.sync_copy(u_v, u_s.at[sl])
        if r == 1:
            pltpu.sync_copy(u_v, u2_hbm.at[sl])
        elif r == 3:
            pltpu.sync_copy(u_v, u4_hbm.at[sl])
        elif r == 5:
            pltpu.sync_copy(u_v, u6_hbm.at[sl])
        if r < 5:
            pltpu.sync_copy(zz_v, acc_s.at[sl])
        plsc.subcore_barrier()


def _prep_body(batch_ref, p0_ref, cind_ref):
    gid = lax.broadcasted_iota(jnp.int32, (1, G), 1)
    onehot = (batch_ref[...] == gid).astype(F32)
    counts = jnp.sum(onehot, axis=0, keepdims=True)
    cmax = jnp.maximum(counts, 1.0)
    p0_ref[...] = onehot / cmax
    cind_ref[...] = counts / cmax


_prep_call = pl.pallas_call(
    _prep_body,
    out_shape=(
        jax.ShapeDtypeStruct((NPAD, G), F32),
        jax.ShapeDtypeStruct((1, G), F32),
    ),
)


def _epi_body(u2_ref, u4_ref, u6_ref, degm1_ref, x_ref, cind_ref,
              W1_ref, b1_ref, W2_ref, b2_ref, W3_ref, b3_ref,
              fcW_ref, fcb_ref, out_ref):
    f32 = dict(preferred_element_type=F32)
    sqd = jnp.sqrt(degm1_ref[...] + 1.0)
    s2 = jnp.sum(sqd * u2_ref[...], axis=0, keepdims=True)
    s4 = jnp.sum(sqd * u4_ref[...], axis=0, keepdims=True)
    q6 = (sqd * u6_ref[...])[:N, :]
    q6x = lax.dot_general(q6, x_ref[...], (((0,), (0,)), ((), ())), **f32)
    W3 = W3_ref[...]
    W32 = jnp.dot(W3, W2_ref[...], **f32)
    Wc = jnp.dot(W32, W1_ref[...], **f32)
    w21 = lax.dot_general(b1_ref[...], W32, (((1,), (1,)), ((), ())), **f32)
    wb2 = lax.dot_general(b2_ref[...], W3, (((1,), (1,)), ((), ())), **f32)
    outer = (((0,), (0,)), ((), ()))
    pooled = (lax.dot_general(q6x, Wc, (((1,), (1,)), ((), ())), **f32)
              + lax.dot_general(s4, w21, outer, **f32)
              + lax.dot_general(s2, wb2, outer, **f32)
              + lax.dot_general(cind_ref[...], b3_ref[...], outer, **f32))
    logits = lax.dot_general(pooled, fcW_ref[...],
                             (((1,), (1,)), ((), ())), **f32) + fcb_ref[...]
    m = jnp.max(logits, axis=1, keepdims=True)
    lse = jnp.log(jnp.sum(jnp.exp(logits - m), axis=1, keepdims=True)) + m
    out_ref[...] = logits - lse


_epi_call = pl.pallas_call(
    _epi_body,
    out_shape=jax.ShapeDtypeStruct((G, 10), F32),
)


def kernel(x, edge_index, batch, W1, b1, W2, b2, W3, b3, fcW, fcb):
    src = edge_index[0]
    dst = edge_index[1]
    pad = N + jnp.arange(EPAD - src.shape[0], dtype=jnp.int32) % (NPAD - N)
    srcg = jnp.concatenate([src, pad]).reshape(NSUB, SCH, SROWS)
    dstg = jnp.concatenate([dst, pad]).reshape(NSUB, SCH, SROWS)
    batchp = jnp.concatenate(
        [batch, jnp.full((NPAD - N,), G, jnp.int32)]).reshape(NPAD, 1)

    p0, cind = _prep_call(batchp)
    u2, u4, u6, degm1 = _prop_kernel(p0, dstg, srcg)
    return _epi_call(u2, u4, u6, degm1, x, cind,
                     W1, b1.reshape(1, -1), W2, b2.reshape(1, -1),
                     W3, b3.reshape(1, -1), fcW, fcb.reshape(1, -1))


# speedup vs baseline: 1.1794x; 1.0296x over previous
import functools

import jax
import jax.numpy as jnp
from jax import lax
from jax.experimental import pallas as pl
from jax.experimental.pallas import tpu as pltpu
from jax.experimental.pallas import tpu_sc as plsc

N = 10000
G = 16
NSUB = 16
ROWS = 632
NPAD = NSUB * ROWS
DUMMY = NPAD - 1
ECH = 160
EPC = ECH * 128
SCW = 1
SCH = ECH // SCW
SROWS = SCW * 128
EPAD = NSUB * EPC
F32 = jnp.float32

_mesh = plsc.VectorSubcoreMesh(
    core_axis_name="c", subcore_axis_name="s", num_cores=1)
_sc_params = pltpu.CompilerParams(
    use_tc_tiling_on_sc=False, needs_layout_passes=False)


@functools.partial(
    pl.kernel,
    out_type=[jax.ShapeDtypeStruct((NPAD, G), F32)] * 4,
    mesh=_mesh,
    compiler_params=_sc_params,
    scratch_types=[
        pltpu.VMEM_SHARED((NPAD, G), F32),
        pltpu.VMEM_SHARED((NPAD, G), F32),
        pltpu.VMEM((SCH, SROWS), jnp.int32),
        pltpu.VMEM((SCH, SROWS), jnp.int32),
        pltpu.VMEM((SROWS, G), F32),
        pltpu.VMEM((SROWS, G), F32),
        pltpu.VMEM((ROWS, G), F32),
        pltpu.VMEM((ROWS, G), F32),
        pltpu.VMEM((ROWS, G), F32),
        pltpu.VMEM((ROWS, G), F32),
        pltpu.VMEM((ROWS + 8, ), jnp.int32),
        pltpu.SemaphoreType.DMA,
        pltpu.SemaphoreType.DMA,
    ],
)
def _prop_kernel(batch_hbm, dst_hbm, src_hbm, u2_hbm, u4_hbm, u6_hbm, deg_hbm,
                 u_s, acc_s, dst_v, src_v, rows_a, rows_b, a_v, u_v, d_v,
                 zz_v, b_v, ga, gb):
    def dsl(j):
        return dst_v.at[j]

    def ssl(j):
        return src_v.at[j]

    w = lax.axis_index("s")
    sl = pl.ds(w * ROWS, ROWS)
    pltpu.sync_copy(dst_hbm.at[w], dst_v)
    pltpu.sync_copy(src_hbm.at[w], src_v)
    zero = jnp.zeros((G,), F32)
    one = jnp.ones((G,), F32)

    @pl.loop(0, ROWS)
    def _(i):
        zz_v[i] = zero

    @pl.loop(0, SROWS)
    def _(i):
        rows_a[i] = one

    pltpu.sync_copy(zz_v, acc_s.at[sl])
    plsc.subcore_barrier()

    @pl.loop(0, SCH)
    def _(j):
        pltpu.async_copy(rows_a, acc_s.at[dsl(j)], ga, add=True)

    @pl.loop(0, SCH)
    def _(j):
        pltpu.make_async_copy(rows_a, acc_s.at[dsl(j)], ga).wait()

    plsc.subcore_barrier()
    pltpu.sync_copy(acc_s.at[sl], a_v)
    pltpu.sync_copy(a_v, deg_hbm.at[sl])
    pltpu.sync_copy(zz_v, acc_s.at[sl])
    pltpu.sync_copy(batch_hbm.at[pl.ds(w * ROWS, ROWS + 8)], b_v)

    @pl.loop(0, ROWS)
    def _(i):
        u_v[i] = zero

    iota16 = lax.iota(jnp.int32, G)
    glim = jnp.full((G,), G, jnp.int32)

    for g in range(ROWS // G + 1):
        b16 = b_v[pl.ds(g * G, G)]
        raw = iota16 + jnp.full((G,), g * G, jnp.int32)
        row16 = jnp.minimum(raw, jnp.full((G,), ROWS - 1, jnp.int32))
        ok = (b16 < glim) & (raw < jnp.full((G,), ROWS, jnp.int32))
        plsc.store_scatter(u_v, [row16, b16], one, mask=ok)

    magic = jnp.full((G,), 0x5F3759DF, jnp.int32)
    shift1 = jnp.full((G,), 1, jnp.int32)
    c15 = jnp.full((G,), 1.5, F32)
    ch = jnp.full((G,), 0.5, F32)

    @pl.loop(0, ROWS)
    def _(i):
        degv = a_v[i] + one
        d_v[i] = one / degv
        y = plsc.bitcast(
            magic - lax.shift_right_logical(plsc.bitcast(degv, jnp.int32),
                                            shift1), F32)
        hx = ch * degv
        y = y * (c15 - hx * y * y)
        y = y * (c15 - hx * y * y)
        y = y * (c15 - hx * y * y)
        u_v[i] = u_v[i] * y

    pltpu.sync_copy(u_v, u_s.at[sl])
    plsc.subcore_barrier()

    for r in range(6):
        pltpu.async_copy(u_s.at[dsl(0)], rows_a, ga)

        @pl.loop(0, SCH - 3, step=2)
        def _(j):
            pltpu.async_copy(u_s.at[dsl(j + 1)], rows_b, gb)
            pltpu.make_async_copy(u_s.at[dsl(j)], rows_a, ga).wait()
            pltpu.sync_copy(rows_a, acc_s.at[ssl(j)], add=True)
            pltpu.async_copy(u_s.at[dsl(j + 2)], rows_a, ga)
            pltpu.make_async_copy(u_s.at[dsl(j + 1)], rows_b, gb).wait()
            pltpu.sync_copy(rows_b, acc_s.at[ssl(j + 1)], add=True)

        pltpu.async_copy(u_s.at[dsl(SCH - 1)], rows_b, gb)
        pltpu.make_async_copy(u_s.at[dsl(SCH - 2)], rows_a, ga).wait()
        pltpu.sync_copy(rows_a, acc_s.at[ssl(SCH - 2)], add=True)
        pltpu.make_async_copy(u_s.at[dsl(SCH - 1)], rows_b, gb).wait()
        pltpu.sync_copy(rows_b, acc_s.at[ssl(SCH - 1)], add=True)

        plsc.subcore_barrier()
        pltpu.sync_copy(acc_s.at[sl], a_v)
        pltpu.sync_copy(u_s.at[sl], u_v)

        @pl.loop(0, ROWS, step=4)
        def _(i):
            for t in range(4):
                u_v[i + t] = d_v[i + t] * (a_v[i + t] + u_v[i + t])

        pltpu.sync_copy(u_v, u_s.at[sl])
        if r == 1:
            pltpu.sync_copy(u_v, u2_hbm.at[sl])
        elif r == 3:
            pltpu.sync_copy(u_v, u4_hbm.at[sl])
        elif r == 5:
            pltpu.sync_copy(u_v, u6_hbm.at[sl])
        if r < 5:
            pltpu.sync_copy(zz_v, acc_s.at[sl])
        plsc.subcore_barrier()


def _epi_body(u2_ref, u4_ref, u6_ref, degm1_ref, x_ref, batch_ref,
              W1_ref, b1_ref, W2_ref, b2_ref, W3_ref, b3_ref,
              fcW_ref, fcb_ref, out_ref):
    f32 = dict(preferred_element_type=F32)
    sqd = jnp.sqrt(degm1_ref[...] + 1.0)
    s2 = jnp.sum(sqd * u2_ref[...], axis=0, keepdims=True)
    s4 = jnp.sum(sqd * u4_ref[...], axis=0, keepdims=True)
    q6 = (sqd * u6_ref[...])[:N, :]
    q6x = lax.dot_general(q6, x_ref[...], (((0,), (0,)), ((), ())), **f32)
    gid = lax.broadcasted_iota(jnp.int32, (1, G), 1)
    onehot = (batch_ref[...] == gid).astype(F32)
    counts_col = lax.dot_general(
        onehot, jnp.ones((NPAD, 1), F32), (((0,), (0,)), ((), ())), **f32)
    cmax_col = jnp.maximum(counts_col, 1.0)
    W3 = W3_ref[...]
    W32 = jnp.dot(W3, W2_ref[...], **f32)
    Wc = jnp.dot(W32, W1_ref[...], **f32)
    w21 = lax.dot_general(b1_ref[...], W32, (((1,), (1,)), ((), ())), **f32)
    wb2 = lax.dot_general(b2_ref[...], W3, (((1,), (1,)), ((), ())), **f32)
    outer = (((0,), (0,)), ((), ()))
    pooled = (lax.dot_general(q6x, Wc, (((1,), (1,)), ((), ())), **f32)
              + lax.dot_general(s4, w21, outer, **f32)
              + lax.dot_general(s2, wb2, outer, **f32)
              + counts_col * b3_ref[...]) / cmax_col
    logits = lax.dot_general(pooled, fcW_ref[...],
                             (((1,), (1,)), ((), ())), **f32) + fcb_ref[...]
    m = jnp.max(logits, axis=1, keepdims=True)
    lse = jnp.log(jnp.sum(jnp.exp(logits - m), axis=1, keepdims=True)) + m
    out_ref[...] = logits - lse


_epi_call = pl.pallas_call(
    _epi_body,
    out_shape=jax.ShapeDtypeStruct((G, 10), F32),
)


def kernel(x, edge_index, batch, W1, b1, W2, b2, W3, b3, fcW, fcb):
    src = edge_index[0]
    dst = edge_index[1]
    pad = N + jnp.arange(EPAD - src.shape[0], dtype=jnp.int32) % (NPAD - N)
    srcg = jnp.concatenate([src, pad]).reshape(NSUB, SCH, SROWS)
    dstg = jnp.concatenate([dst, pad]).reshape(NSUB, SCH, SROWS)
    batchp = jnp.concatenate(
        [batch, jnp.full((NPAD + 128 - N,), G, jnp.int32)])

    u2, u4, u6, degm1 = _prop_kernel(batchp, dstg, srcg)
    return _epi_call(u2, u4, u6, degm1, x, batchp[:NPAD].reshape(NPAD, 1),
                     W1, b1.reshape(1, -1), W2, b2.reshape(1, -1),
                     W3, b3.reshape(1, -1), fcW, fcb.reshape(1, -1))


# speedup vs baseline: 1.1927x; 1.0113x over previous
import functools

import jax
import jax.numpy as jnp
from jax import lax
from jax.experimental import pallas as pl
from jax.experimental.pallas import tpu as pltpu
from jax.experimental.pallas import tpu_sc as plsc

N = 10000
G = 16
NSUB = 16
ROWS = 632
NPAD = NSUB * ROWS
DUMMY = NPAD - 1
ECH = 160
EPC = ECH * 128
SCW = 1
SCH = ECH // SCW
SROWS = SCW * 128
EPAD = NSUB * EPC
F32 = jnp.float32

_mesh = plsc.VectorSubcoreMesh(
    core_axis_name="c", subcore_axis_name="s", num_cores=1)
_sc_params = pltpu.CompilerParams(
    use_tc_tiling_on_sc=False, needs_layout_passes=False)


@functools.partial(
    pl.kernel,
    out_type=[jax.ShapeDtypeStruct((NPAD, G), F32)] * 4,
    mesh=_mesh,
    compiler_params=_sc_params,
    scratch_types=[
        pltpu.VMEM_SHARED((NPAD, G), F32),
        pltpu.VMEM_SHARED((NPAD, G), F32),
        pltpu.VMEM((SCH, SROWS), jnp.int32),
        pltpu.VMEM((SCH, SROWS), jnp.int32),
        pltpu.VMEM((SROWS, G), F32),
        pltpu.VMEM((SROWS, G), F32),
        pltpu.VMEM((ROWS, G), F32),
        pltpu.VMEM((ROWS, G), F32),
        pltpu.VMEM((ROWS, G), F32),
        pltpu.VMEM((ROWS, G), F32),
        pltpu.VMEM((ROWS + 8, ), jnp.int32),
        pltpu.SemaphoreType.DMA,
        pltpu.SemaphoreType.DMA,
        pltpu.SemaphoreType.DMA,
    ],
)
def _prop_kernel(batch_hbm, dst_hbm, src_hbm, u2_hbm, u4_hbm, u6_hbm, deg_hbm,
                 u_s, acc_s, dst_v, src_v, rows_a, rows_b, a_v, u_v, d_v,
                 zz_v, b_v, ga, gb, sn):
    def dsl(j):
        return dst_v.at[j]

    def ssl(j):
        return src_v.at[j]

    w = lax.axis_index("s")
    sl = pl.ds(w * ROWS, ROWS)
    pltpu.sync_copy(dst_hbm.at[w], dst_v)
    pltpu.sync_copy(src_hbm.at[w], src_v)
    zero = jnp.zeros((G,), F32)
    one = jnp.ones((G,), F32)

    @pl.loop(0, ROWS)
    def _(i):
        zz_v[i] = zero

    @pl.loop(0, SROWS)
    def _(i):
        rows_a[i] = one

    pltpu.sync_copy(zz_v, acc_s.at[sl])
    plsc.subcore_barrier()

    @pl.loop(0, SCH)
    def _(j):
        pltpu.async_copy(rows_a, acc_s.at[dsl(j)], ga, add=True)

    @pl.loop(0, SCH)
    def _(j):
        pltpu.make_async_copy(rows_a, acc_s.at[dsl(j)], ga).wait()

    plsc.subcore_barrier()
    pltpu.sync_copy(acc_s.at[sl], a_v)
    pltpu.sync_copy(a_v, deg_hbm.at[sl])
    pltpu.sync_copy(zz_v, acc_s.at[sl])
    pltpu.sync_copy(batch_hbm.at[pl.ds(w * ROWS, ROWS + 8)], b_v)

    @pl.loop(0, ROWS)
    def _(i):
        u_v[i] = zero

    iota16 = lax.iota(jnp.int32, G)
    glim = jnp.full((G,), G, jnp.int32)

    for g in range(ROWS // G + 1):
        b16 = b_v[pl.ds(g * G, G)]
        raw = iota16 + jnp.full((G,), g * G, jnp.int32)
        row16 = jnp.minimum(raw, jnp.full((G,), ROWS - 1, jnp.int32))
        ok = (b16 < glim) & (raw < jnp.full((G,), ROWS, jnp.int32))
        plsc.store_scatter(u_v, [row16, b16], one, mask=ok)

    magic = jnp.full((G,), 0x5F3759DF, jnp.int32)
    shift1 = jnp.full((G,), 1, jnp.int32)
    c15 = jnp.full((G,), 1.5, F32)
    ch = jnp.full((G,), 0.5, F32)

    @pl.loop(0, ROWS)
    def _(i):
        degv = a_v[i] + one
        d_v[i] = one / degv
        y = plsc.bitcast(
            magic - lax.shift_right_logical(plsc.bitcast(degv, jnp.int32),
                                            shift1), F32)
        hx = ch * degv
        y = y * (c15 - hx * y * y)
        y = y * (c15 - hx * y * y)
        y = y * (c15 - hx * y * y)
        u_v[i] = u_v[i] * y

    pltpu.sync_copy(u_v, u_s.at[sl])
    plsc.subcore_barrier()

    for r in range(6):
        pltpu.async_copy(u_s.at[dsl(0)], rows_a, ga)

        @pl.loop(0, SCH - 3, step=2)
        def _(j):
            pltpu.async_copy(u_s.at[dsl(j + 1)], rows_b, gb)
            pltpu.make_async_copy(u_s.at[dsl(j)], rows_a, ga).wait()
            pltpu.sync_copy(rows_a, acc_s.at[ssl(j)], add=True)
            pltpu.async_copy(u_s.at[dsl(j + 2)], rows_a, ga)
            pltpu.make_async_copy(u_s.at[dsl(j + 1)], rows_b, gb).wait()
            pltpu.sync_copy(rows_b, acc_s.at[ssl(j + 1)], add=True)

        pltpu.async_copy(u_s.at[dsl(SCH - 1)], rows_b, gb)
        pltpu.make_async_copy(u_s.at[dsl(SCH - 2)], rows_a, ga).wait()
        pltpu.sync_copy(rows_a, acc_s.at[ssl(SCH - 2)], add=True)
        pltpu.make_async_copy(u_s.at[dsl(SCH - 1)], rows_b, gb).wait()
        pltpu.sync_copy(rows_b, acc_s.at[ssl(SCH - 1)], add=True)

        plsc.subcore_barrier()
        pltpu.sync_copy(acc_s.at[sl], a_v)
        if r == 2:
            pltpu.make_async_copy(u_v, u2_hbm.at[sl], sn).wait()
        elif r == 4:
            pltpu.make_async_copy(u_v, u4_hbm.at[sl], sn).wait()

        @pl.loop(0, ROWS, step=4)
        def _(i):
            for t in range(4):
                u_v[i + t] = d_v[i + t] * (a_v[i + t] + u_v[i + t])

        pltpu.sync_copy(u_v, u_s.at[sl])
        if r == 1:
            pltpu.async_copy(u_v, u2_hbm.at[sl], sn)
        elif r == 3:
            pltpu.async_copy(u_v, u4_hbm.at[sl], sn)
        elif r == 5:
            pltpu.sync_copy(u_v, u6_hbm.at[sl])
        if r < 5:
            pltpu.sync_copy(zz_v, acc_s.at[sl])
        plsc.subcore_barrier()


def _epi_body(u2_ref, u4_ref, u6_ref, degm1_ref, x_ref, batch_ref,
              W1_ref, b1_ref, W2_ref, b2_ref, W3_ref, b3_ref,
              fcW_ref, fcb_ref, out_ref):
    f32 = dict(preferred_element_type=F32)
    sqd = jnp.sqrt(degm1_ref[...] + 1.0)
    s2 = jnp.sum(sqd * u2_ref[...], axis=0, keepdims=True)
    s4 = jnp.sum(sqd * u4_ref[...], axis=0, keepdims=True)
    q6 = (sqd * u6_ref[...])[:N, :]
    q6x = lax.dot_general(q6, x_ref[...], (((0,), (0,)), ((), ())), **f32)
    gid = lax.broadcasted_iota(jnp.int32, (1, G), 1)
    onehot = (batch_ref[...] == gid).astype(F32)
    counts_col = lax.dot_general(
        onehot, jnp.ones((NPAD, 1), F32), (((0,), (0,)), ((), ())), **f32)
    cmax_col = jnp.maximum(counts_col, 1.0)
    W3 = W3_ref[...]
    W32 = jnp.dot(W3, W2_ref[...], **f32)
    Wc = jnp.dot(W32, W1_ref[...], **f32)
    w21 = lax.dot_general(b1_ref[...], W32, (((1,), (1,)), ((), ())), **f32)
    wb2 = lax.dot_general(b2_ref[...], W3, (((1,), (1,)), ((), ())), **f32)
    outer = (((0,), (0,)), ((), ()))
    pooled = (lax.dot_general(q6x, Wc, (((1,), (1,)), ((), ())), **f32)
              + lax.dot_general(s4, w21, outer, **f32)
              + lax.dot_general(s2, wb2, outer, **f32)
              + counts_col * b3_ref[...]) / cmax_col
    logits = lax.dot_general(pooled, fcW_ref[...],
                             (((1,), (1,)), ((), ())), **f32) + fcb_ref[...]
    m = jnp.max(logits, axis=1, keepdims=True)
    lse = jnp.log(jnp.sum(jnp.exp(logits - m), axis=1, keepdims=True)) + m
    out_ref[...] = logits - lse


_epi_call = pl.pallas_call(
    _epi_body,
    out_shape=jax.ShapeDtypeStruct((G, 10), F32),
)


def kernel(x, edge_index, batch, W1, b1, W2, b2, W3, b3, fcW, fcb):
    src = edge_index[0]
    dst = edge_index[1]
    pad = N + jnp.arange(EPAD - src.shape[0], dtype=jnp.int32) % (NPAD - N)
    srcg = jnp.concatenate([src, pad]).reshape(NSUB, SCH, SROWS)
    dstg = jnp.concatenate([dst, pad]).reshape(NSUB, SCH, SROWS)
    batchp = jnp.concatenate(
        [batch, jnp.full((NPAD + 128 - N,), G, jnp.int32)])

    u2, u4, u6, degm1 = _prop_kernel(batchp, dstg, srcg)
    return _epi_call(u2, u4, u6, degm1, x, batchp[:NPAD].reshape(NPAD, 1),
                     W1, b1.reshape(1, -1), W2, b2.reshape(1, -1),
                     W3, b3.reshape(1, -1), fcW, fcb.reshape(1, -1))


# speedup vs baseline: 1.2638x; 1.0596x over previous
import functools

import jax
import jax.numpy as jnp
from jax import lax
from jax.experimental import pallas as pl
from jax.experimental.pallas import tpu as pltpu
from jax.experimental.pallas import tpu_sc as plsc

N = 10000
G = 16
NSUB = 16
ROWS = 632
NPAD = NSUB * ROWS
DUMMY = NPAD - 1
ECH = 160
EPC = ECH * 128
SCW = 2
SCH = ECH // SCW
SROWS = SCW * 128
EPAD = NSUB * EPC
F32 = jnp.float32

_mesh = plsc.VectorSubcoreMesh(
    core_axis_name="c", subcore_axis_name="s", num_cores=1)
_sc_params = pltpu.CompilerParams(
    use_tc_tiling_on_sc=False, needs_layout_passes=False)


@functools.partial(
    pl.kernel,
    out_type=[jax.ShapeDtypeStruct((NPAD, G), F32)] * 4,
    mesh=_mesh,
    compiler_params=_sc_params,
    scratch_types=[
        pltpu.VMEM_SHARED((NPAD, G), F32),
        pltpu.VMEM_SHARED((NPAD, G), F32),
        pltpu.VMEM((SCH, SROWS), jnp.int32),
        pltpu.VMEM((SCH, SROWS), jnp.int32),
        pltpu.VMEM((SROWS, G), F32),
        pltpu.VMEM((SROWS, G), F32),
        pltpu.VMEM((ROWS, G), F32),
        pltpu.VMEM((ROWS, G), F32),
        pltpu.VMEM((ROWS, G), F32),
        pltpu.VMEM((ROWS, G), F32),
        pltpu.VMEM((ROWS + 8, ), jnp.int32),
        pltpu.SemaphoreType.DMA,
        pltpu.SemaphoreType.DMA,
        pltpu.SemaphoreType.DMA,
    ],
)
def _prop_kernel(batch_hbm, dst_hbm, src_hbm, u2_hbm, u4_hbm, u6_hbm, deg_hbm,
                 u_s, acc_s, dst_v, src_v, rows_a, rows_b, a_v, u_v, d_v,
                 zz_v, b_v, ga, gb, sn):
    def dsl(j):
        return dst_v.at[j]

    def ssl(j):
        return src_v.at[j]

    w = lax.axis_index("s")
    sl = pl.ds(w * ROWS, ROWS)
    pltpu.sync_copy(dst_hbm.at[w], dst_v)
    pltpu.sync_copy(src_hbm.at[w], src_v)
    zero = jnp.zeros((G,), F32)
    one = jnp.ones((G,), F32)

    @pl.loop(0, ROWS)
    def _(i):
        zz_v[i] = zero

    @pl.loop(0, SROWS)
    def _(i):
        rows_a[i] = one

    pltpu.sync_copy(zz_v, acc_s.at[sl])
    plsc.subcore_barrier()

    @pl.loop(0, SCH)
    def _(j):
        pltpu.async_copy(rows_a, acc_s.at[dsl(j)], ga, add=True)

    @pl.loop(0, SCH)
    def _(j):
        pltpu.make_async_copy(rows_a, acc_s.at[dsl(j)], ga).wait()

    plsc.subcore_barrier()
    pltpu.sync_copy(acc_s.at[sl], a_v)
    pltpu.sync_copy(a_v, deg_hbm.at[sl])
    pltpu.sync_copy(zz_v, acc_s.at[sl])
    pltpu.sync_copy(batch_hbm.at[pl.ds(w * ROWS, ROWS + 8)], b_v)

    @pl.loop(0, ROWS)
    def _(i):
        u_v[i] = zero

    iota16 = lax.iota(jnp.int32, G)
    glim = jnp.full((G,), G, jnp.int32)

    for g in range(ROWS // G + 1):
        b16 = b_v[pl.ds(g * G, G)]
        raw = iota16 + jnp.full((G,), g * G, jnp.int32)
        row16 = jnp.minimum(raw, jnp.full((G,), ROWS - 1, jnp.int32))
        ok = (b16 < glim) & (raw < jnp.full((G,), ROWS, jnp.int32))
        plsc.store_scatter(u_v, [row16, b16], one, mask=ok)

    magic = jnp.full((G,), 0x5F3759DF, jnp.int32)
    shift1 = jnp.full((G,), 1, jnp.int32)
    c15 = jnp.full((G,), 1.5, F32)
    ch = jnp.full((G,), 0.5, F32)

    @pl.loop(0, ROWS)
    def _(i):
        degv = a_v[i] + one
        d_v[i] = one / degv
        y = plsc.bitcast(
            magic - lax.shift_right_logical(plsc.bitcast(degv, jnp.int32),
                                            shift1), F32)
        hx = ch * degv
        y = y * (c15 - hx * y * y)
        y = y * (c15 - hx * y * y)
        y = y * (c15 - hx * y * y)
        u_v[i] = u_v[i] * y

    pltpu.sync_copy(u_v, u_s.at[sl])
    plsc.subcore_barrier()

    for r in range(6):
        pltpu.async_copy(u_s.at[dsl(0)], rows_a, ga)

        @pl.loop(0, SCH - 3, step=2)
        def _(j):
            pltpu.async_copy(u_s.at[dsl(j + 1)], rows_b, gb)
            pltpu.make_async_copy(u_s.at[dsl(j)], rows_a, ga).wait()
            pltpu.sync_copy(rows_a, acc_s.at[ssl(j)], add=True)
            pltpu.async_copy(u_s.at[dsl(j + 2)], rows_a, ga)
            pltpu.make_async_copy(u_s.at[dsl(j + 1)], rows_b, gb).wait()
            pltpu.sync_copy(rows_b, acc_s.at[ssl(j + 1)], add=True)

        pltpu.async_copy(u_s.at[dsl(SCH - 1)], rows_b, gb)
        pltpu.make_async_copy(u_s.at[dsl(SCH - 2)], rows_a, ga).wait()
        pltpu.sync_copy(rows_a, acc_s.at[ssl(SCH - 2)], add=True)
        pltpu.make_async_copy(u_s.at[dsl(SCH - 1)], rows_b, gb).wait()
        pltpu.sync_copy(rows_b, acc_s.at[ssl(SCH - 1)], add=True)

        plsc.subcore_barrier()
        pltpu.sync_copy(acc_s.at[sl], a_v)
        if r == 2:
            pltpu.make_async_copy(u_v, u2_hbm.at[sl], sn).wait()
        elif r == 4:
            pltpu.make_async_copy(u_v, u4_hbm.at[sl], sn).wait()

        @pl.loop(0, ROWS, step=4)
        def _(i):
            for t in range(4):
                u_v[i + t] = d_v[i + t] * (a_v[i + t] + u_v[i + t])

        pltpu.sync_copy(u_v, u_s.at[sl])
        if r == 1:
            pltpu.async_copy(u_v, u2_hbm.at[sl], sn)
        elif r == 3:
            pltpu.async_copy(u_v, u4_hbm.at[sl], sn)
        elif r == 5:
            pltpu.sync_copy(u_v, u6_hbm.at[sl])
        if r < 5:
            pltpu.sync_copy(zz_v, acc_s.at[sl])
        plsc.subcore_barrier()


def _epi_body(u2_ref, u4_ref, u6_ref, degm1_ref, x_ref, batch_ref,
              W1_ref, b1_ref, W2_ref, b2_ref, W3_ref, b3_ref,
              fcW_ref, fcb_ref, out_ref):
    f32 = dict(preferred_element_type=F32)
    sqd = jnp.sqrt(degm1_ref[...] + 1.0)
    s2 = jnp.sum(sqd * u2_ref[...], axis=0, keepdims=True)
    s4 = jnp.sum(sqd * u4_ref[...], axis=0, keepdims=True)
    q6 = (sqd * u6_ref[...])[:N, :]
    q6x = lax.dot_general(q6, x_ref[...], (((0,), (0,)), ((), ())), **f32)
    gid = lax.broadcasted_iota(jnp.int32, (1, G), 1)
    onehot = (batch_ref[...] == gid).astype(F32)
    counts_col = lax.dot_general(
        onehot, jnp.ones((NPAD, 1), F32), (((0,), (0,)), ((), ())), **f32)
    cmax_col = jnp.maximum(counts_col, 1.0)
    W3 = W3_ref[...]
    W32 = jnp.dot(W3, W2_ref[...], **f32)
    Wc = jnp.dot(W32, W1_ref[...], **f32)
    w21 = lax.dot_general(b1_ref[...], W32, (((1,), (1,)), ((), ())), **f32)
    wb2 = lax.dot_general(b2_ref[...], W3, (((1,), (1,)), ((), ())), **f32)
    outer = (((0,), (0,)), ((), ()))
    pooled = (lax.dot_general(q6x, Wc, (((1,), (1,)), ((), ())), **f32)
              + lax.dot_general(s4, w21, outer, **f32)
              + lax.dot_general(s2, wb2, outer, **f32)
              + counts_col * b3_ref[...]) / cmax_col
    logits = lax.dot_general(pooled, fcW_ref[...],
                             (((1,), (1,)), ((), ())), **f32) + fcb_ref[...]
    m = jnp.max(logits, axis=1, keepdims=True)
    lse = jnp.log(jnp.sum(jnp.exp(logits - m), axis=1, keepdims=True)) + m
    out_ref[...] = logits - lse


_epi_call = pl.pallas_call(
    _epi_body,
    out_shape=jax.ShapeDtypeStruct((G, 10), F32),
)


def kernel(x, edge_index, batch, W1, b1, W2, b2, W3, b3, fcW, fcb):
    src = edge_index[0]
    dst = edge_index[1]
    pad = N + jnp.arange(EPAD - src.shape[0], dtype=jnp.int32) % (NPAD - N)
    srcg = jnp.concatenate([src, pad]).reshape(NSUB, SCH, SROWS)
    dstg = jnp.concatenate([dst, pad]).reshape(NSUB, SCH, SROWS)
    batchp = jnp.concatenate(
        [batch, jnp.full((NPAD + 128 - N,), G, jnp.int32)])

    u2, u4, u6, degm1 = _prop_kernel(batchp, dstg, srcg)
    return _epi_call(u2, u4, u6, degm1, x, batchp[:NPAD].reshape(NPAD, 1),
                     W1, b1.reshape(1, -1), W2, b2.reshape(1, -1),
                     W3, b3.reshape(1, -1), fcW, fcb.reshape(1, -1))
